# Initial kernel scaffold; baseline (speedup 1.0000x reference)
import jax
import jax.numpy as jnp
from jax.experimental import pallas as pl


def kernel(inputs):
    raise NotImplementedError("write your pallas kernel here")


# speedup vs baseline: 1.7787x; 1.7787x over previous
import functools

import jax
import jax.numpy as jnp
from jax import lax
from jax.experimental import pallas as pl
from jax.experimental.pallas import tpu as pltpu
from jax.experimental.pallas import tpu_sc as plsc

N_ROWS = 16384
D = 64
L = 16
NUM_CORES = 2
NUM_SUBCORES = 16
NW = NUM_CORES * NUM_SUBCORES
ROWS_PER_W = N_ROWS // NW
THRESHOLD = 0.01


def _row_topk_softmax(x):
  m01 = jnp.maximum(x[0], x[1])
  m23 = jnp.maximum(x[2], x[3])
  m = jnp.max(jnp.maximum(m01, m23))
  e = [jnp.exp(v - m) for v in x]
  s = jnp.sum((e[0] + e[1]) + (e[2] + e[3]))
  inv = jnp.full((L,), 1.0, jnp.float32) / jnp.broadcast_to(s, (L,))
  p = [v * inv for v in e]
  s0 = lax.sort(p[0])
  s1 = lax.sort(p[1])
  s2 = lax.sort(p[2])
  s3 = lax.sort(p[3])
  h1 = jnp.maximum(s0, lax.rev(s1, (0,)))
  h2 = jnp.maximum(s2, lax.rev(s3, (0,)))
  h = jnp.maximum(lax.sort(h1), lax.rev(lax.sort(h2), (0,)))
  hs = lax.sort(h)
  lane = lax.iota(jnp.int32, L)
  t8 = jnp.max(jnp.where(lane == 8, hs, -jnp.inf))
  thr = jnp.maximum(t8, jnp.float32(THRESHOLD))
  return [jnp.where(v >= thr, v, jnp.float32(0.0)) for v in p]


def _body(x_hbm, out_hbm, in_v, out_v):
  wid = lax.axis_index("s") * NUM_CORES + lax.axis_index("c")
  base = wid * ROWS_PER_W
  pltpu.sync_copy(x_hbm.at[pl.ds(base, ROWS_PER_W)], in_v)

  def row_step(r):
    x = [in_v[r, pl.ds(16 * j, L)] for j in range(4)]
    o = _row_topk_softmax(x)
    for j in range(4):
      out_v[r, pl.ds(16 * j, L)] = o[j]

  plsc.parallel_loop(0, ROWS_PER_W, 1, unroll=1)(row_step)

  pltpu.sync_copy(out_v, out_hbm.at[pl.ds(base, ROWS_PER_W)])


@jax.jit
def kernel(inputs):
  mesh = plsc.VectorSubcoreMesh(core_axis_name="c", subcore_axis_name="s")
  f = pl.kernel(
      _body,
      out_type=jax.ShapeDtypeStruct((N_ROWS, D), jnp.float32),
      mesh=mesh,
      scratch_types=[
          pltpu.VMEM((ROWS_PER_W, D), jnp.float32),
          pltpu.VMEM((ROWS_PER_W, D), jnp.float32),
      ],
      compiler_params=pltpu.CompilerParams(needs_layout_passes=False),
  )
  return f(inputs)


# speedup vs baseline: 1.8232x; 1.0251x over previous
import functools

import jax
import jax.numpy as jnp
from jax import lax
from jax.experimental import pallas as pl
from jax.experimental.pallas import tpu as pltpu
from jax.experimental.pallas import tpu_sc as plsc

N_ROWS = 16384
D = 64
L = 16
NUM_CORES = 2
NUM_SUBCORES = 16
NW = NUM_CORES * NUM_SUBCORES
ROWS_PER_W = N_ROWS // NW
THRESHOLD = 0.01


def _row_topk_softmax(x):
  s0 = lax.sort(x[0])
  s1 = lax.sort(x[1])
  s2 = lax.sort(x[2])
  s3 = lax.sort(x[3])
  h1 = jnp.maximum(s0, lax.rev(s1, (0,)))
  h2 = jnp.maximum(s2, lax.rev(s3, (0,)))
  h = jnp.maximum(lax.sort(h1), lax.rev(lax.sort(h2), (0,)))
  hs = lax.sort(h)
  m = hs[jnp.full((L,), 15, jnp.int32)]
  t8 = hs[jnp.full((L,), 8, jnp.int32)]
  e = [jnp.exp(v - m) for v in x]
  s = jnp.sum((e[0] + e[1]) + (e[2] + e[3]))
  inv = jnp.full((L,), 1.0, jnp.float32) / jnp.broadcast_to(s, (L,))
  p = [v * inv for v in e]
  thr = jnp.float32(THRESHOLD)
  return [
      jnp.where((v >= t8) & (q >= thr), q, jnp.float32(0.0))
      for v, q in zip(x, p)
  ]


def _body(x_hbm, out_hbm, in_v, out_v):
  wid = lax.axis_index("s") * NUM_CORES + lax.axis_index("c")
  base = wid * ROWS_PER_W
  pltpu.sync_copy(x_hbm.at[pl.ds(base, ROWS_PER_W)], in_v)

  def row_step(r):
    x = [in_v[r, pl.ds(16 * j, L)] for j in range(4)]
    o = _row_topk_softmax(x)
    for j in range(4):
      out_v[r, pl.ds(16 * j, L)] = o[j]

  plsc.parallel_loop(0, ROWS_PER_W, 1, unroll=2)(row_step)

  pltpu.sync_copy(out_v, out_hbm.at[pl.ds(base, ROWS_PER_W)])


@jax.jit
def kernel(inputs):
  mesh = plsc.VectorSubcoreMesh(core_axis_name="c", subcore_axis_name="s")
  f = pl.kernel(
      _body,
      out_type=jax.ShapeDtypeStruct((N_ROWS, D), jnp.float32),
      mesh=mesh,
      scratch_types=[
          pltpu.VMEM((ROWS_PER_W, D), jnp.float32),
          pltpu.VMEM((ROWS_PER_W, D), jnp.float32),
      ],
      compiler_params=pltpu.CompilerParams(needs_layout_passes=False),
  )
  return f(inputs)


# speedup vs baseline: 1.8235x; 1.0001x over previous
import functools

import jax
import jax.numpy as jnp
from jax import lax
from jax.experimental import pallas as pl
from jax.experimental.pallas import tpu as pltpu
from jax.experimental.pallas import tpu_sc as plsc

N_ROWS = 16384
D = 64
L = 16
NUM_CORES = 2
NUM_SUBCORES = 16
NW = NUM_CORES * NUM_SUBCORES
ROWS_PER_W = N_ROWS // NW
THRESHOLD = 0.01


def _row_topk_softmax(x):
  s0 = lax.sort(x[0])
  s1 = lax.sort(x[1])
  s2 = lax.sort(x[2])
  s3 = lax.sort(x[3])
  h1 = jnp.maximum(s0, lax.rev(s1, (0,)))
  h2 = jnp.maximum(s2, lax.rev(s3, (0,)))
  h = jnp.maximum(lax.sort(h1), lax.rev(lax.sort(h2), (0,)))
  hs = lax.sort(h)
  m = hs[jnp.full((L,), 15, jnp.int32)]
  t8 = hs[jnp.full((L,), 8, jnp.int32)]
  e = [jnp.exp(v - m) for v in x]
  s = jnp.sum((e[0] + e[1]) + (e[2] + e[3]))
  inv = jnp.full((L,), 1.0, jnp.float32) / jnp.broadcast_to(s, (L,))
  p = [v * inv for v in e]
  thr = jnp.float32(THRESHOLD)
  return [
      jnp.where((v >= t8) & (q >= thr), q, jnp.float32(0.0))
      for v, q in zip(x, p)
  ]


def _body(x_hbm, out_hbm, in_v, out_v):
  wid = lax.axis_index("s") * NUM_CORES + lax.axis_index("c")
  base = wid * ROWS_PER_W
  pltpu.sync_copy(x_hbm.at[pl.ds(base, ROWS_PER_W)], in_v)

  def row_step(r):
    x = [in_v[r, pl.ds(16 * j, L)] for j in range(4)]
    o = _row_topk_softmax(x)
    for j in range(4):
      out_v[r, pl.ds(16 * j, L)] = o[j]

  plsc.parallel_loop(0, ROWS_PER_W, 1, unroll=2)(row_step)

  pltpu.sync_copy(out_v, out_hbm.at[pl.ds(base, ROWS_PER_W)])


@jax.jit
def kernel(inputs):
  mesh = plsc.VectorSubcoreMesh(core_axis_name="c", subcore_axis_name="s")
  f = pl.kernel(
      _body,
      out_type=jax.ShapeDtypeStruct((N_ROWS, D), jnp.float32),
      mesh=mesh,
      scratch_types=[
          pltpu.VMEM((ROWS_PER_W, D), jnp.float32),
          pltpu.VMEM((ROWS_PER_W, D), jnp.float32),
      ],
      compiler_params=pltpu.CompilerParams(needs_layout_passes=False, use_tc_tiling_on_sc=True),
  )
  return f(inputs)
